# baseline (speedup 1.0000x reference)
import jax
import jax.numpy as jnp
from jax import lax
from jax.experimental import pallas as pl
from jax.experimental.pallas import tpu as pltpu

_S = 2
_S3 = _S ** 3
_D = 128


def _body(scale_ref, coords_ref, feats_ref, coords_out_ref, feats_out_ref):
    f = feats_ref[...]
    B, d = f.shape
    feats_out_ref[...] = jnp.broadcast_to(f[:, None, :], (B, _S3, d))

    c16 = coords_ref[...]
    s = scale_ref[0]
    a = lax.broadcasted_iota(jnp.int32, (16, 128), 0)
    l = lax.broadcasted_iota(jnp.int32, (16, 128), 1)
    k = l & 3
    sel = ((l >> 5) * 4 + k) == a
    mult = jnp.where(k == 0, 1, s)
    msel = jnp.where(sel, mult, 0).astype(jnp.float32)
    lr = lax.broadcasted_iota(jnp.int32, (1, 128), 1)
    kr = lr & 3
    jr = (lr >> 2) & 7
    off = jnp.where(
        kr == 0, 0,
        jnp.where(kr == 1, (jr >> 2) & 1,
                  jnp.where(kr == 2, (jr >> 1) & 1, jr & 1)))
    prod = jax.lax.dot_general(
        c16.astype(jnp.float32), msel, (((1,), (0,)), ((), ())),
        preferred_element_type=jnp.float32)
    coords_out_ref[...] = prod.astype(jnp.int32) + off


def kernel(coords, feats, scale):
    N, d = feats.shape
    B = 2048
    grid = ((N + B - 1) // B,)
    scale_arr = jnp.asarray(scale, jnp.int32).reshape(1)
    coords16 = coords.reshape(N // 4, 16)
    coords_out, feats_out = pl.pallas_call(
        _body,
        grid=grid,
        in_specs=[
            pl.BlockSpec(memory_space=pltpu.SMEM),
            pl.BlockSpec((B // 4, 16), lambda i: (i, 0)),
            pl.BlockSpec((B, d), lambda i: (i, 0)),
        ],
        out_specs=[
            pl.BlockSpec((B // 4, 128), lambda i: (i, 0)),
            pl.BlockSpec((B, _S3, d), lambda i: (i, 0, 0)),
        ],
        out_shape=[
            jax.ShapeDtypeStruct((N // 4, 128), jnp.int32),
            jax.ShapeDtypeStruct((N, _S3, d), jnp.float32),
        ],
    )(scale_arr, coords16, feats)
    return coords_out.reshape(N * _S3, 4), feats_out.reshape(N * _S3, d)


# speedup vs baseline: 1.5739x; 1.5739x over previous
import jax
import jax.numpy as jnp
from jax import lax
from jax.experimental import pallas as pl
from jax.experimental.pallas import tpu as pltpu

_S = 2
_S3 = _S ** 3
_D = 128


def _body(scale_ref, coords_ref, feats_ref, coords_out_ref, feats_out_ref):
    f = feats_ref[...]
    B, d = f.shape
    feats_out_ref[...] = jnp.broadcast_to(f[:, None, :], (B, _S3, d))

    c = coords_ref[...]
    s = scale_ref[0]
    c_rep = lax.broadcast_in_dim(c, (B, _S3, 4), (0, 2)).reshape(B * _S3, 4)
    r = lax.broadcasted_iota(jnp.int32, (B * _S3, 4), 0)
    k = lax.broadcasted_iota(jnp.int32, (B * _S3, 4), 1)
    j = r & 7
    mult = jnp.where(k == 0, 1, s)
    off = jnp.where(
        k == 0, 0,
        jnp.where(k == 1, (j >> 2) & 1,
                  jnp.where(k == 2, (j >> 1) & 1, j & 1)))
    coords_out_ref[...] = c_rep * mult + off


def kernel(coords, feats, scale):
    N, d = feats.shape
    B = 2000
    grid = (N // B,)
    scale_arr = jnp.asarray(scale, jnp.int32).reshape(1)
    coords_out, feats_out = pl.pallas_call(
        _body,
        grid=grid,
        in_specs=[
            pl.BlockSpec(memory_space=pltpu.SMEM),
            pl.BlockSpec((B, 4), lambda i: (i, 0)),
            pl.BlockSpec((B, d), lambda i: (i, 0)),
        ],
        out_specs=[
            pl.BlockSpec((B * _S3, 4), lambda i: (i, 0)),
            pl.BlockSpec((B, _S3, d), lambda i: (i, 0, 0)),
        ],
        out_shape=[
            jax.ShapeDtypeStruct((N * _S3, 4), jnp.int32),
            jax.ShapeDtypeStruct((N, _S3, d), jnp.float32),
        ],
    )(scale_arr, coords, feats)
    return coords_out, feats_out.reshape(N * _S3, d)
